# baseline (speedup 1.0000x reference)
import functools

import jax
import jax.numpy as jnp
from jax import lax
from jax.experimental import pallas as pl
from jax.experimental.pallas import tpu as pltpu
from jax.experimental.pallas import tpu_sc as plsc

N_NODES = 10000
N_EDGES = 320000
D = 128

NC = 2
NS = 16
NW = NC * NS
NP = 10240
CHUNK = 128
C0 = 24
C1 = 133
CTOT = C0 + C1
E_PAD = NS * CTOT * CHUNK
ROWS_PER_S = NP // NS
ZROWS = 128


def _seg_body(support_hbm, sdidx_hbm, part_hbm,
              acc, idx_v, rows_v, isem, gsem):
    c = lax.axis_index("c")
    s = lax.axis_index("s")
    wid = s * NC + c

    def zrow(r, carry):
        for j in range(D // 16):
            rows_v[0, r, pl.ds(j * 16, 16)] = jnp.zeros((16,), jnp.float32)
        return carry
    lax.fori_loop(0, CHUNK, zrow, 0)
    for t in range(ROWS_PER_S // CHUNK):
        pltpu.sync_copy(rows_v.at[0],
                        acc.at[pl.ds(s * ROWS_PER_S + t * CHUNK, CHUNK)])
    plsc.subcore_barrier()

    col0 = c * C0
    nch = jnp.where(c == 0, C0, C1)

    def idx_load(i, r):
        pltpu.async_copy(sdidx_hbm.at[s, col0 + i], idx_v.at[r], isem)

    def idx_wait(i, r):
        pltpu.make_async_copy(sdidx_hbm.at[s, col0 + i], idx_v.at[r],
                              isem).wait()

    def gather(r, b):
        pltpu.async_copy(support_hbm.at[idx_v.at[r, 0]], rows_v.at[b], gsem)

    def gather_wait(r, b):
        pltpu.make_async_copy(support_hbm.at[idx_v.at[r, 0]],
                              rows_v.at[b], gsem).wait()

    idx_load(0, 0)
    idx_load(1, 1)
    idx_wait(0, 0)
    gather(0, 0)

    def chunk(i, carry):
        b = lax.rem(i, 2)
        r = lax.rem(i, 4)

        @pl.when(i + 2 < nch)
        def _():
            idx_load(i + 2, lax.rem(i + 2, 4))

        @pl.when(i + 1 < nch)
        def _():
            idx_wait(i + 1, lax.rem(i + 1, 4))
            gather(lax.rem(i + 1, 4), 1 - b)

        gather_wait(r, b)
        pltpu.sync_copy(rows_v.at[b], acc.at[idx_v.at[r, 1]], add=True)
        return carry
    lax.fori_loop(0, nch, chunk, 0)

    plsc.subcore_barrier()
    for t in range(ROWS_PER_S // ZROWS):
        r0 = s * ROWS_PER_S + t * ZROWS
        pltpu.sync_copy(acc.at[pl.ds(r0, ZROWS)], part_hbm.at[c, pl.ds(r0, ZROWS)])


_segment_sum_sc = functools.partial(
    pl.kernel,
    out_type=jax.ShapeDtypeStruct((NC, NP, D), jnp.float32),
    mesh=plsc.VectorSubcoreMesh(core_axis_name="c", subcore_axis_name="s",
                                num_cores=NC, num_subcores=NS),
    scratch_types=[
        pltpu.VMEM_SHARED((NP, D), jnp.float32),
        pltpu.VMEM((4, 2, CHUNK), jnp.int32),
        pltpu.VMEM((2, CHUNK, D), jnp.float32),
        pltpu.SemaphoreType.DMA,
        pltpu.SemaphoreType.DMA,
    ],
)(_seg_body)


def _mm_body(x_ref, w_ref, o_ref):
    o_ref[...] = jnp.dot(x_ref[...], w_ref[...],
                         preferred_element_type=jnp.float32)


def _fused_mm_body(p_ref, b_ref, w_ref, o_ref):
    xb = jnp.maximum(p_ref[0] + p_ref[1] + b_ref[...], 0.0)
    o_ref[...] = jnp.dot(xb, w_ref[...], preferred_element_type=jnp.float32)


def _relu_out_body(p_ref, b_ref, o_ref):
    o_ref[...] = jnp.maximum(p_ref[0] + p_ref[1] + b_ref[...], 0.0)


_RB = 1024


def _mm(x, w):
    return pl.pallas_call(
        _mm_body,
        grid=(NP // _RB,),
        in_specs=[pl.BlockSpec((_RB, D), lambda i: (i, 0)),
                  pl.BlockSpec((D, D), lambda i: (0, 0))],
        out_specs=pl.BlockSpec((_RB, D), lambda i: (i, 0)),
        out_shape=jax.ShapeDtypeStruct((NP, D), jnp.float32),
    )(x, w)


def _fused_mm(part, b2d, w):
    return pl.pallas_call(
        _fused_mm_body,
        grid=(NP // _RB,),
        in_specs=[pl.BlockSpec((NC, _RB, D), lambda i: (0, i, 0)),
                  pl.BlockSpec((1, D), lambda i: (0, 0)),
                  pl.BlockSpec((D, D), lambda i: (0, 0))],
        out_specs=pl.BlockSpec((_RB, D), lambda i: (i, 0)),
        out_shape=jax.ShapeDtypeStruct((NP, D), jnp.float32),
    )(part, b2d, w)


def _relu_out(part, b2d):
    return pl.pallas_call(
        _relu_out_body,
        grid=(NP // _RB,),
        in_specs=[pl.BlockSpec((NC, _RB, D), lambda i: (0, i, 0)),
                  pl.BlockSpec((1, D), lambda i: (0, 0))],
        out_specs=pl.BlockSpec((_RB, D), lambda i: (i, 0)),
        out_shape=jax.ShapeDtypeStruct((NP, D), jnp.float32),
    )(part, b2d)


def kernel(mesh, shape_features, W0, b0, W1, b1, W2, b2):
    npad = E_PAD - N_EDGES
    src = jnp.concatenate(
        [mesh[0].astype(jnp.int32), jnp.zeros((npad,), jnp.int32)])
    dst = jnp.concatenate(
        [mesh[1].astype(jnp.int32),
         N_NODES + (jnp.arange(npad, dtype=jnp.int32) % (NP - N_NODES))])
    sdidx = jnp.stack([src.reshape(NS, CTOT, CHUNK),
                       dst.reshape(NS, CTOT, CHUNK)], axis=2)

    x = jnp.pad(shape_features, ((0, NP - N_NODES), (0, 0)))
    support = _mm(x, W0)
    part = _segment_sum_sc(support, sdidx)
    support = _fused_mm(part, b0.reshape(1, D), W1)
    part = _segment_sum_sc(support, sdidx)
    support = _fused_mm(part, b1.reshape(1, D), W2)
    part = _segment_sum_sc(support, sdidx)
    return _relu_out(part, b2.reshape(1, D))[:N_NODES]


# speedup vs baseline: 1.0346x; 1.0346x over previous
import functools

import jax
import jax.numpy as jnp
from jax import lax
from jax.experimental import pallas as pl
from jax.experimental.pallas import tpu as pltpu
from jax.experimental.pallas import tpu_sc as plsc

N_NODES = 10000
N_EDGES = 320000
D = 128

NC = 2
NS = 16
NW = NC * NS
NP = 10240
CHUNK = 128
C0 = 32
C1 = 125
CTOT = C0 + C1
E_PAD = NS * CTOT * CHUNK
ROWS_PER_S = NP // NS
ZROWS = 128


def _seg_body(support_hbm, sdidx_hbm, part_hbm,
              acc, idx_v, rows_v, isem, gsem):
    c = lax.axis_index("c")
    s = lax.axis_index("s")
    wid = s * NC + c

    def zrow(r, carry):
        for j in range(D // 16):
            rows_v[0, r, pl.ds(j * 16, 16)] = jnp.zeros((16,), jnp.float32)
        return carry
    lax.fori_loop(0, CHUNK, zrow, 0)
    for t in range(ROWS_PER_S // CHUNK):
        pltpu.async_copy(rows_v.at[0],
                         acc.at[pl.ds(s * ROWS_PER_S + t * CHUNK, CHUNK)],
                         isem)
    for t in range(ROWS_PER_S // CHUNK):
        pltpu.make_async_copy(rows_v.at[0],
                              acc.at[pl.ds(s * ROWS_PER_S + t * CHUNK, CHUNK)],
                              isem).wait()
    plsc.subcore_barrier()

    col0 = c * C0
    nch = jnp.where(c == 0, C0, C1)

    def idx_load(i, r):
        pltpu.async_copy(sdidx_hbm.at[s, col0 + i], idx_v.at[r], isem)

    def idx_wait(i, r):
        pltpu.make_async_copy(sdidx_hbm.at[s, col0 + i], idx_v.at[r],
                              isem).wait()

    def gather(r, b):
        pltpu.async_copy(support_hbm.at[idx_v.at[r, 0]], rows_v.at[b], gsem)

    def gather_wait(r, b):
        pltpu.make_async_copy(support_hbm.at[idx_v.at[r, 0]],
                              rows_v.at[b], gsem).wait()

    idx_load(0, 0)
    idx_load(1, 1)
    idx_wait(0, 0)
    gather(0, 0)

    def chunk(i, carry):
        b = lax.rem(i, 2)
        r = lax.rem(i, 4)

        @pl.when(i + 2 < nch)
        def _():
            idx_load(i + 2, lax.rem(i + 2, 4))

        @pl.when(i + 1 < nch)
        def _():
            idx_wait(i + 1, lax.rem(i + 1, 4))
            gather(lax.rem(i + 1, 4), 1 - b)

        gather_wait(r, b)
        pltpu.sync_copy(rows_v.at[b], acc.at[idx_v.at[r, 1]], add=True)
        return carry
    lax.fori_loop(0, nch, chunk, 0)

    plsc.subcore_barrier()
    for t in range(ROWS_PER_S // ZROWS):
        r0 = s * ROWS_PER_S + t * ZROWS
        pltpu.async_copy(acc.at[pl.ds(r0, ZROWS)],
                         part_hbm.at[c, pl.ds(r0, ZROWS)], isem)
    for t in range(ROWS_PER_S // ZROWS):
        r0 = s * ROWS_PER_S + t * ZROWS
        pltpu.make_async_copy(acc.at[pl.ds(r0, ZROWS)],
                              part_hbm.at[c, pl.ds(r0, ZROWS)], isem).wait()


_segment_sum_sc = functools.partial(
    pl.kernel,
    out_type=jax.ShapeDtypeStruct((NC, NP, D), jnp.float32),
    mesh=plsc.VectorSubcoreMesh(core_axis_name="c", subcore_axis_name="s",
                                num_cores=NC, num_subcores=NS),
    scratch_types=[
        pltpu.VMEM_SHARED((NP, D), jnp.float32),
        pltpu.VMEM((4, 2, CHUNK), jnp.int32),
        pltpu.VMEM((2, CHUNK, D), jnp.float32),
        pltpu.SemaphoreType.DMA,
        pltpu.SemaphoreType.DMA,
    ],
)(_seg_body)


def _mm_body(x_ref, w_ref, o_ref):
    o_ref[...] = jnp.dot(x_ref[...], w_ref[...],
                         preferred_element_type=jnp.float32)


def _fused_mm_body(p_ref, b_ref, w_ref, o_ref):
    xb = jnp.maximum(p_ref[0] + p_ref[1] + b_ref[...], 0.0)
    o_ref[...] = jnp.dot(xb, w_ref[...], preferred_element_type=jnp.float32)


def _relu_out_body(p_ref, b_ref, o_ref):
    o_ref[...] = jnp.maximum(p_ref[0] + p_ref[1] + b_ref[...], 0.0)


_RB = 1024


def _mm(x, w):
    return pl.pallas_call(
        _mm_body,
        grid=(NP // _RB,),
        in_specs=[pl.BlockSpec((_RB, D), lambda i: (i, 0)),
                  pl.BlockSpec((D, D), lambda i: (0, 0))],
        out_specs=pl.BlockSpec((_RB, D), lambda i: (i, 0)),
        out_shape=jax.ShapeDtypeStruct((NP, D), jnp.float32),
    )(x, w)


def _fused_mm(part, b2d, w):
    return pl.pallas_call(
        _fused_mm_body,
        grid=(NP // _RB,),
        in_specs=[pl.BlockSpec((NC, _RB, D), lambda i: (0, i, 0)),
                  pl.BlockSpec((1, D), lambda i: (0, 0)),
                  pl.BlockSpec((D, D), lambda i: (0, 0))],
        out_specs=pl.BlockSpec((_RB, D), lambda i: (i, 0)),
        out_shape=jax.ShapeDtypeStruct((NP, D), jnp.float32),
    )(part, b2d, w)


def _relu_out(part, b2d):
    return pl.pallas_call(
        _relu_out_body,
        grid=(NP // _RB,),
        in_specs=[pl.BlockSpec((NC, _RB, D), lambda i: (0, i, 0)),
                  pl.BlockSpec((1, D), lambda i: (0, 0))],
        out_specs=pl.BlockSpec((_RB, D), lambda i: (i, 0)),
        out_shape=jax.ShapeDtypeStruct((NP, D), jnp.float32),
    )(part, b2d)


def kernel(mesh, shape_features, W0, b0, W1, b1, W2, b2):
    npad = E_PAD - N_EDGES
    src = jnp.concatenate(
        [mesh[0].astype(jnp.int32), jnp.zeros((npad,), jnp.int32)])
    dst = jnp.concatenate(
        [mesh[1].astype(jnp.int32),
         N_NODES + (jnp.arange(npad, dtype=jnp.int32) % (NP - N_NODES))])
    sdidx = jnp.stack([src.reshape(NS, CTOT, CHUNK),
                       dst.reshape(NS, CTOT, CHUNK)], axis=2)

    x = jnp.pad(shape_features, ((0, NP - N_NODES), (0, 0)))
    support = _mm(x, W0)
    part = _segment_sum_sc(support, sdidx)
    support = _fused_mm(part, b0.reshape(1, D), W1)
    part = _segment_sum_sc(support, sdidx)
    support = _fused_mm(part, b1.reshape(1, D), W2)
    part = _segment_sum_sc(support, sdidx)
    return _relu_out(part, b2.reshape(1, D))[:N_NODES]


# speedup vs baseline: 1.0369x; 1.0022x over previous
import functools

import jax
import jax.numpy as jnp
from jax import lax
from jax.experimental import pallas as pl
from jax.experimental.pallas import tpu as pltpu
from jax.experimental.pallas import tpu_sc as plsc

N_NODES = 10000
N_EDGES = 320000
D = 128

NC = 2
NS = 16
NP = 10240
CHUNK = 128
C0 = 32
C1 = 125
CTOT = C0 + C1
E_PAD = NS * CTOT * CHUNK
ROWS_PER_S = NP // NS
ZROWS = 128


def _seg_body(support_hbm, sdidx_hbm, part_hbm,
              acc, idx_v, rows_v, isem, gsem):
    c = lax.axis_index("c")
    s = lax.axis_index("s")

    def zrow(r, carry):
        for j in range(D // 16):
            rows_v[0, r, pl.ds(j * 16, 16)] = jnp.zeros((16,), jnp.float32)
        return carry
    lax.fori_loop(0, CHUNK, zrow, 0)
    for t in range(ROWS_PER_S // CHUNK):
        pltpu.async_copy(rows_v.at[0],
                         acc.at[pl.ds(s * ROWS_PER_S + t * CHUNK, CHUNK)],
                         isem)
    for t in range(ROWS_PER_S // CHUNK):
        pltpu.make_async_copy(rows_v.at[0],
                              acc.at[pl.ds(s * ROWS_PER_S + t * CHUNK, CHUNK)],
                              isem).wait()
    plsc.subcore_barrier()

    col0 = c * C0
    nch = jnp.where(c == 0, C0, C1)

    def idx_load(i, r):
        pltpu.async_copy(sdidx_hbm.at[s, col0 + i], idx_v.at[r], isem)

    def idx_wait(i, r):
        pltpu.make_async_copy(sdidx_hbm.at[s, col0 + i], idx_v.at[r],
                              isem).wait()

    def gather(r, b):
        pltpu.async_copy(support_hbm.at[idx_v.at[r, 0]], rows_v.at[b], gsem)

    def gather_wait(r, b):
        pltpu.make_async_copy(support_hbm.at[idx_v.at[r, 0]],
                              rows_v.at[b], gsem).wait()

    idx_load(0, 0)
    idx_load(1, 1)
    idx_wait(0, 0)
    gather(0, 0)

    def chunk(i, carry):
        b = lax.rem(i, 2)
        r = lax.rem(i, 4)

        @pl.when(i + 2 < nch)
        def _():
            idx_load(i + 2, lax.rem(i + 2, 4))

        @pl.when(i + 1 < nch)
        def _():
            idx_wait(i + 1, lax.rem(i + 1, 4))
            gather(lax.rem(i + 1, 4), 1 - b)

        gather_wait(r, b)
        pltpu.sync_copy(rows_v.at[b], acc.at[idx_v.at[r, 1]], add=True)
        return carry
    lax.fori_loop(0, nch, chunk, 0)

    plsc.subcore_barrier()
    for t in range(ROWS_PER_S // ZROWS):
        r0 = s * ROWS_PER_S + t * ZROWS
        pltpu.async_copy(acc.at[pl.ds(r0, ZROWS)],
                         part_hbm.at[c, pl.ds(r0, ZROWS)], isem)
    for t in range(ROWS_PER_S // ZROWS):
        r0 = s * ROWS_PER_S + t * ZROWS
        pltpu.make_async_copy(acc.at[pl.ds(r0, ZROWS)],
                              part_hbm.at[c, pl.ds(r0, ZROWS)], isem).wait()


_segment_sum_sc = functools.partial(
    pl.kernel,
    out_type=jax.ShapeDtypeStruct((NC, NP, D), jnp.float32),
    mesh=plsc.VectorSubcoreMesh(core_axis_name="c", subcore_axis_name="s",
                                num_cores=NC, num_subcores=NS),
    scratch_types=[
        pltpu.VMEM_SHARED((NP, D), jnp.float32),
        pltpu.VMEM((4, 2, CHUNK), jnp.int32),
        pltpu.VMEM((2, CHUNK, D), jnp.float32),
        pltpu.SemaphoreType.DMA,
        pltpu.SemaphoreType.DMA,
    ],
)(_seg_body)


def _mm_body(x_ref, w_ref, o_ref):
    o_ref[...] = jnp.dot(x_ref[...], w_ref[...],
                         preferred_element_type=jnp.float32)


def _fused_mm_body(p_ref, b_ref, w_ref, o_ref):
    xb = jnp.maximum(p_ref[0] + p_ref[1] + b_ref[...], 0.0)
    o_ref[...] = jnp.dot(xb, w_ref[...], preferred_element_type=jnp.float32)


def _relu_out_body(p_ref, b_ref, o_ref):
    o_ref[...] = jnp.maximum(p_ref[0] + p_ref[1] + b_ref[...], 0.0)


_RB = 1024


def _mm(x, w):
    return pl.pallas_call(
        _mm_body,
        grid=(NP // _RB,),
        in_specs=[pl.BlockSpec((_RB, D), lambda i: (i, 0)),
                  pl.BlockSpec((D, D), lambda i: (0, 0))],
        out_specs=pl.BlockSpec((_RB, D), lambda i: (i, 0)),
        out_shape=jax.ShapeDtypeStruct((NP, D), jnp.float32),
    )(x, w)


def _fused_mm(part, b2d, w):
    return pl.pallas_call(
        _fused_mm_body,
        grid=(NP // _RB,),
        in_specs=[pl.BlockSpec((NC, _RB, D), lambda i: (0, i, 0)),
                  pl.BlockSpec((1, D), lambda i: (0, 0)),
                  pl.BlockSpec((D, D), lambda i: (0, 0))],
        out_specs=pl.BlockSpec((_RB, D), lambda i: (i, 0)),
        out_shape=jax.ShapeDtypeStruct((NP, D), jnp.float32),
    )(part, b2d, w)


def _relu_out(part, b2d):
    return pl.pallas_call(
        _relu_out_body,
        grid=(NP // _RB,),
        in_specs=[pl.BlockSpec((NC, _RB, D), lambda i: (0, i, 0)),
                  pl.BlockSpec((1, D), lambda i: (0, 0))],
        out_specs=pl.BlockSpec((_RB, D), lambda i: (i, 0)),
        out_shape=jax.ShapeDtypeStruct((NP, D), jnp.float32),
    )(part, b2d)


def kernel(mesh, shape_features, W0, b0, W1, b1, W2, b2):
    npad = E_PAD - N_EDGES
    src = jnp.concatenate(
        [mesh[0].astype(jnp.int32), jnp.zeros((npad,), jnp.int32)])
    dst = jnp.concatenate(
        [mesh[1].astype(jnp.int32),
         N_NODES + (jnp.arange(npad, dtype=jnp.int32) % (NP - N_NODES))])
    sdidx = jnp.stack([src.reshape(NS, CTOT, CHUNK),
                       dst.reshape(NS, CTOT, CHUNK)], axis=2)

    x = jnp.pad(shape_features, ((0, NP - N_NODES), (0, 0)))
    support = _mm(x, W0)
    part = _segment_sum_sc(support, sdidx)
    support = _fused_mm(part, b0.reshape(1, D), W1)
    part = _segment_sum_sc(support, sdidx)
    support = _fused_mm(part, b1.reshape(1, D), W2)
    part = _segment_sum_sc(support, sdidx)
    return _relu_out(part, b2.reshape(1, D))[:N_NODES]
